# Initial kernel scaffold; baseline (speedup 1.0000x reference)
import jax
import jax.numpy as jnp
from jax.experimental import pallas as pl


def kernel(labels, q):
    raise NotImplementedError("write your pallas kernel here")


# speedup vs baseline: 1.7465x; 1.7465x over previous
import functools
import jax
import jax.numpy as jnp
from jax import lax
from jax.experimental import pallas as pl
from jax.experimental.pallas import tpu as pltpu
from jax.experimental.pallas import tpu_sc as plsc

SIZE_PER_CLS = 32
N_CLASSES = 100
EMB = 64
NEG_PER_CLS = 8
BS = 512
PERM = (31, 7, 4, 29, 16, 19, 2, 5)

NC = 2
NS = 16
NW = NC * NS
B_PER_W = BS // NW
NEG_C = N_CLASSES - 1


def _sc_body(labels_hbm, q2_hbm, full_hbm, pos_hbm, neg_hbm,
             full_v, lbl_v, pos_v, sem_g, sem_n):
    wid = lax.axis_index("s") * NC + lax.axis_index("c")
    base = wid * B_PER_W

    pltpu.sync_copy(labels_hbm.at[pl.ds(base, B_PER_W)], lbl_v)
    gcp = pltpu.make_async_copy(q2_hbm.at[lbl_v], pos_v, sem_g)
    gcp.start()
    pltpu.sync_copy(full_hbm, full_v)

    lblvec = lbl_v[...]
    for r in range(B_PER_W):
        lb = lblvec[r]

        def fire(j, c, lb=lb, r=r):
            src = j + (j >= lb).astype(jnp.int32)
            pltpu.make_async_copy(
                full_v.at[:, src, :],
                neg_hbm.at[base + r, :, pl.ds(j * NEG_PER_CLS, NEG_PER_CLS)],
                sem_n).start()
            return c

        def drain(j, c, r=r):
            pltpu.make_async_copy(
                full_v.at[:, 0, :],
                neg_hbm.at[base + r, :, pl.ds(0, NEG_PER_CLS)],
                sem_n).wait()
            return c

        lax.fori_loop(0, NEG_C, fire, 0)
        lax.fori_loop(0, NEG_C, drain, 0)

    gcp.wait()
    pltpu.sync_copy(pos_v, pos_hbm.at[pl.ds(base, B_PER_W)])


_sc_call = functools.partial(
    pl.kernel,
    mesh=plsc.VectorSubcoreMesh(core_axis_name="c", subcore_axis_name="s"),
    out_type=(
        jax.ShapeDtypeStruct((BS, EMB * SIZE_PER_CLS), jnp.float32),
        jax.ShapeDtypeStruct((BS, EMB, NEG_C * NEG_PER_CLS), jnp.float32),
    ),
    scratch_types=[
        pltpu.VMEM((EMB, N_CLASSES, NEG_PER_CLS), jnp.float32),
        pltpu.VMEM((B_PER_W,), jnp.int32),
        pltpu.VMEM((B_PER_W, EMB * SIZE_PER_CLS), jnp.float32),
        pltpu.SemaphoreType.DMA,
        pltpu.SemaphoreType.DMA,
    ],
    compiler_params=pltpu.CompilerParams(use_tc_tiling_on_sc=False),
)(_sc_body)


def kernel(labels, q):
    labels = labels.astype(jnp.int32)
    q_sel = jnp.stack([q[:, :, p] for p in PERM], axis=-1)
    full = jnp.transpose(q_sel, (1, 0, 2))
    q2 = q.reshape(N_CLASSES, EMB * SIZE_PER_CLS)
    pos, neg = _sc_call(labels, q2, full)
    return pos.reshape(BS, EMB, SIZE_PER_CLS), neg


# speedup vs baseline: 2.2987x; 1.3162x over previous
import functools
import jax
import jax.numpy as jnp
from jax import lax
from jax.experimental import pallas as pl
from jax.experimental.pallas import tpu as pltpu
from jax.experimental.pallas import tpu_sc as plsc

SIZE_PER_CLS = 32
N_CLASSES = 100
EMB = 64
NEG_PER_CLS = 8
BS = 512
PERM = (31, 7, 4, 29, 16, 19, 2, 5)

NC = 2
NS = 16
NW = NC * NS
B_PER_W = BS // NW
NEG_C = N_CLASSES - 1


def _sc_body(labels_hbm, q2_hbm, full_hbm, pos_hbm, neg_hbm,
             full_v, lbl_v, pos_v, sem_g, sem_n, sem_big):
    wid = lax.axis_index("s") * NC + lax.axis_index("c")
    base = wid * B_PER_W

    pltpu.sync_copy(labels_hbm.at[pl.ds(base, B_PER_W)], lbl_v)
    gcp = pltpu.make_async_copy(q2_hbm.at[lbl_v], pos_v, sem_g)
    gcp.start()
    pltpu.sync_copy(full_hbm, full_v)

    big_copies = []
    for r in range(B_PER_W):
        cp = pltpu.make_async_copy(
            full_v.at[:, pl.ds(0, NEG_C * NEG_PER_CLS)],
            neg_hbm.at[base + r],
            sem_big.at[r])
        cp.start()
        big_copies.append(cp)

    lblvec = lbl_v[...]
    for r in range(B_PER_W):
        big_copies[r].wait()
        lb = lblvec[r]

        def fire(j, c, r=r):
            pltpu.make_async_copy(
                full_v.at[:, pl.ds((j + 1) * NEG_PER_CLS, NEG_PER_CLS)],
                neg_hbm.at[base + r, :, pl.ds(j * NEG_PER_CLS, NEG_PER_CLS)],
                sem_n).start()
            return c

        lax.fori_loop(lb, NEG_C, fire, 0)

    lbl_sum = lblvec[0]
    for r in range(1, B_PER_W):
        lbl_sum = lbl_sum + lblvec[r]
    total = B_PER_W * NEG_C - lbl_sum

    def drain(i, c):
        pltpu.make_async_copy(
            full_v.at[:, pl.ds(0, NEG_PER_CLS)],
            neg_hbm.at[base, :, pl.ds(0, NEG_PER_CLS)],
            sem_n).wait()
        return c

    lax.fori_loop(0, total, drain, 0)

    gcp.wait()
    pltpu.sync_copy(pos_v, pos_hbm.at[pl.ds(base, B_PER_W)])


_sc_call = functools.partial(
    pl.kernel,
    mesh=plsc.VectorSubcoreMesh(core_axis_name="c", subcore_axis_name="s"),
    out_type=(
        jax.ShapeDtypeStruct((BS, EMB * SIZE_PER_CLS), jnp.float32),
        jax.ShapeDtypeStruct((BS, EMB, NEG_C * NEG_PER_CLS), jnp.float32),
    ),
    scratch_types=[
        pltpu.VMEM((EMB, N_CLASSES * NEG_PER_CLS), jnp.float32),
        pltpu.VMEM((B_PER_W,), jnp.int32),
        pltpu.VMEM((B_PER_W, EMB * SIZE_PER_CLS), jnp.float32),
        pltpu.SemaphoreType.DMA,
        pltpu.SemaphoreType.DMA,
        pltpu.SemaphoreType.DMA((B_PER_W,)),
    ],
    compiler_params=pltpu.CompilerParams(use_tc_tiling_on_sc=False),
)(_sc_body)


def kernel(labels, q):
    labels = labels.astype(jnp.int32)
    q_sel = jnp.stack([q[:, :, p] for p in PERM], axis=-1)
    full = jnp.transpose(q_sel, (1, 0, 2)).reshape(EMB, N_CLASSES * NEG_PER_CLS)
    q2 = q.reshape(N_CLASSES, EMB * SIZE_PER_CLS)
    pos, neg = _sc_call(labels, q2, full)
    return pos.reshape(BS, EMB, SIZE_PER_CLS), neg


# speedup vs baseline: 4.3405x; 1.8882x over previous
import functools
import jax
import jax.numpy as jnp
from jax import lax
from jax.experimental import pallas as pl
from jax.experimental.pallas import tpu as pltpu
from jax.experimental.pallas import tpu_sc as plsc

SIZE_PER_CLS = 32
N_CLASSES = 100
EMB = 64
NEG_PER_CLS = 8
BS = 512
PERM = (31, 7, 4, 29, 16, 19, 2, 5)

NC = 2
NS = 16
NW = NC * NS
B_PER_W = BS // NW
NEG_C = N_CLASSES - 1


NEG_W = NEG_C * NEG_PER_CLS


def _sc_body(labels_hbm, q2_hbm, full_hbm, pos_hbm, neg_hbm,
             full_v, lbl_v, pos_v, sem_g, sem_n):
    wid = lax.axis_index("s") * NC + lax.axis_index("c")
    base = wid * B_PER_W

    pltpu.sync_copy(labels_hbm.at[pl.ds(base, B_PER_W)],
                    lbl_v.at[pl.ds(0, B_PER_W)])
    gcp = pltpu.make_async_copy(
        q2_hbm.at[lbl_v.at[pl.ds(0, B_PER_W)]], pos_v, sem_g)
    gcp.start()
    pltpu.sync_copy(full_hbm, full_v)

    def row_fire(r, c):
        lv = lbl_v[pl.ds(r, 16)]
        lb = lv[0]
        k = lax.shift_right_logical(lb, 3)

        for kk in range(13):
            @pl.when(k == kk)
            def _(kk=kk, r=r):
                if kk > 0:
                    w = 64 * kk
                    pltpu.make_async_copy(
                        full_v.at[:, pl.ds(0, w)],
                        neg_hbm.at[base + r, :, pl.ds(0, w)],
                        sem_n).start()
                if kk < 12:
                    w = 728 - 64 * kk
                    pltpu.make_async_copy(
                        full_v.at[:, pl.ds(64 * kk + 72, w)],
                        neg_hbm.at[base + r, :, pl.ds(64 * kk + 64, w)],
                        sem_n).start()

        for i in range(NEG_PER_CLS):
            j = k * NEG_PER_CLS + i

            @pl.when(j < NEG_C)
            def _(j=j, r=r, lb=lb):
                src = j + (j >= lb).astype(jnp.int32)
                pltpu.make_async_copy(
                    full_v.at[:, pl.ds(src * NEG_PER_CLS, NEG_PER_CLS)],
                    neg_hbm.at[base + r, :,
                               pl.ds(j * NEG_PER_CLS, NEG_PER_CLS)],
                    sem_n).start()

        return c

    lax.fori_loop(0, B_PER_W, row_fire, 0)

    def row_drain(r, c):
        pltpu.make_async_copy(
            full_v.at[:, pl.ds(0, NEG_W)],
            neg_hbm.at[base + r],
            sem_n).wait()
        return c

    lax.fori_loop(0, B_PER_W, row_drain, 0)

    gcp.wait()
    pltpu.sync_copy(pos_v, pos_hbm.at[pl.ds(base, B_PER_W)])


_sc_call = functools.partial(
    pl.kernel,
    mesh=plsc.VectorSubcoreMesh(core_axis_name="c", subcore_axis_name="s"),
    out_type=(
        jax.ShapeDtypeStruct((BS, EMB * SIZE_PER_CLS), jnp.float32),
        jax.ShapeDtypeStruct((BS, EMB, NEG_C * NEG_PER_CLS), jnp.float32),
    ),
    scratch_types=[
        pltpu.VMEM((EMB, N_CLASSES * NEG_PER_CLS), jnp.float32),
        pltpu.VMEM((B_PER_W + 16,), jnp.int32),
        pltpu.VMEM((B_PER_W, EMB * SIZE_PER_CLS), jnp.float32),
        pltpu.SemaphoreType.DMA,
        pltpu.SemaphoreType.DMA,
    ],
    compiler_params=pltpu.CompilerParams(use_tc_tiling_on_sc=False),
)(_sc_body)


def kernel(labels, q):
    labels = labels.astype(jnp.int32)
    q_sel = jnp.stack([q[:, :, p] for p in PERM], axis=-1)
    full = jnp.transpose(q_sel, (1, 0, 2)).reshape(EMB, N_CLASSES * NEG_PER_CLS)
    q2 = q.reshape(N_CLASSES, EMB * SIZE_PER_CLS)
    pos, neg = _sc_call(labels, q2, full)
    return pos.reshape(BS, EMB, SIZE_PER_CLS), neg


# speedup vs baseline: 4.8768x; 1.1236x over previous
import functools
import jax
import jax.numpy as jnp
from jax import lax
from jax.experimental import pallas as pl
from jax.experimental.pallas import tpu as pltpu
from jax.experimental.pallas import tpu_sc as plsc

SIZE_PER_CLS = 32
N_CLASSES = 100
EMB = 64
NEG_PER_CLS = 8
BS = 512
PERM = (31, 7, 4, 29, 16, 19, 2, 5)

NC = 2
NS = 16
NW = NC * NS
B_PER_W = BS // NW
NEG_C = N_CLASSES - 1


NEG_W = NEG_C * NEG_PER_CLS


def _sc_body(labels_hbm, q2_hbm, full_hbm, pos_hbm, neg_hbm,
             full_v, lbl_v, pos_v, sem_g, sem_n):
    wid = lax.axis_index("s") * NC + lax.axis_index("c")
    base = wid * B_PER_W

    pltpu.sync_copy(labels_hbm.at[pl.ds(base, B_PER_W)],
                    lbl_v.at[pl.ds(0, B_PER_W)])
    gcp = pltpu.make_async_copy(
        q2_hbm.at[lbl_v.at[pl.ds(0, B_PER_W)]], pos_v, sem_g)
    gcp.start()
    pltpu.sync_copy(full_hbm, full_v)

    def row_fire(r, c):
        lv = lbl_v[pl.ds(r, 16)]
        lb = lv[0]

        for kk in range(N_CLASSES):
            @pl.when(lb == kk)
            def _(kk=kk, r=r):
                if kk > 0:
                    w = NEG_PER_CLS * kk
                    pltpu.make_async_copy(
                        full_v.at[:, pl.ds(0, w)],
                        neg_hbm.at[base + r, :, pl.ds(0, w)],
                        sem_n).start()
                if kk < NEG_C:
                    w = NEG_W - NEG_PER_CLS * kk
                    pltpu.make_async_copy(
                        full_v.at[:, pl.ds(NEG_PER_CLS * kk + NEG_PER_CLS, w)],
                        neg_hbm.at[base + r, :, pl.ds(NEG_PER_CLS * kk, w)],
                        sem_n).start()

        return c

    lax.fori_loop(0, B_PER_W, row_fire, 0)

    def row_drain(r, c):
        pltpu.make_async_copy(
            full_v.at[:, pl.ds(0, NEG_W)],
            neg_hbm.at[base + r],
            sem_n).wait()
        return c

    lax.fori_loop(0, B_PER_W, row_drain, 0)

    gcp.wait()
    pltpu.sync_copy(pos_v, pos_hbm.at[pl.ds(base, B_PER_W)])


_sc_call = functools.partial(
    pl.kernel,
    mesh=plsc.VectorSubcoreMesh(core_axis_name="c", subcore_axis_name="s"),
    out_type=(
        jax.ShapeDtypeStruct((BS, EMB * SIZE_PER_CLS), jnp.float32),
        jax.ShapeDtypeStruct((BS, EMB, NEG_C * NEG_PER_CLS), jnp.float32),
    ),
    scratch_types=[
        pltpu.VMEM((EMB, N_CLASSES * NEG_PER_CLS), jnp.float32),
        pltpu.VMEM((B_PER_W + 16,), jnp.int32),
        pltpu.VMEM((B_PER_W, EMB * SIZE_PER_CLS), jnp.float32),
        pltpu.SemaphoreType.DMA,
        pltpu.SemaphoreType.DMA,
    ],
    compiler_params=pltpu.CompilerParams(use_tc_tiling_on_sc=False),
)(_sc_body)


def kernel(labels, q):
    labels = labels.astype(jnp.int32)
    q_sel = jnp.stack([q[:, :, p] for p in PERM], axis=-1)
    full = jnp.transpose(q_sel, (1, 0, 2)).reshape(EMB, N_CLASSES * NEG_PER_CLS)
    q2 = q.reshape(N_CLASSES, EMB * SIZE_PER_CLS)
    pos, neg = _sc_call(labels, q2, full)
    return pos.reshape(BS, EMB, SIZE_PER_CLS), neg


# speedup vs baseline: 4.8778x; 1.0002x over previous
import functools
import jax
import jax.numpy as jnp
from jax import lax
from jax.experimental import pallas as pl
from jax.experimental.pallas import tpu as pltpu
from jax.experimental.pallas import tpu_sc as plsc

SIZE_PER_CLS = 32
N_CLASSES = 100
EMB = 64
NEG_PER_CLS = 8
BS = 512
PERM = (31, 7, 4, 29, 16, 19, 2, 5)

NC = 2
NS = 16
NW = NC * NS
B_PER_W = BS // NW
NEG_C = N_CLASSES - 1


NEG_W = NEG_C * NEG_PER_CLS


def _sc_body(labels_hbm, q2_hbm, full_hbm, pos_hbm, neg_hbm,
             full_v, lbl_v, pos_v, sem_g, sem_n):
    wid = lax.axis_index("s") * NC + lax.axis_index("c")
    base = wid * B_PER_W

    pltpu.sync_copy(labels_hbm.at[pl.ds(base, B_PER_W)],
                    lbl_v.at[pl.ds(0, B_PER_W)])
    gcp = pltpu.make_async_copy(
        q2_hbm.at[lbl_v.at[pl.ds(0, B_PER_W)]], pos_v, sem_g)
    gcp.start()
    pltpu.sync_copy(full_hbm, full_v)

    def row_fire(r, c):
        lv = lbl_v[pl.ds(r, 16)]
        lb = lv[0]
        off_row = (base + r) * (EMB * NEG_W)

        for kk in range(N_CLASSES):
            @pl.when(lb == kk)
            def _(kk=kk, off_row=off_row):
                wa = NEG_PER_CLS * kk
                wb = NEG_W - NEG_PER_CLS * kk

                def per_e(e, cc):
                    off = off_row + e * NEG_W
                    if kk > 0:
                        pltpu.make_async_copy(
                            full_v.at[e, pl.ds(0, wa)],
                            neg_hbm.at[pl.ds(off, wa)],
                            sem_n).start()
                    if kk < NEG_C:
                        pltpu.make_async_copy(
                            full_v.at[e, pl.ds(wa + NEG_PER_CLS, wb)],
                            neg_hbm.at[pl.ds(off + wa, wb)],
                            sem_n).start()
                    return cc

                lax.fori_loop(0, EMB, per_e, 0)

        return c

    lax.fori_loop(0, B_PER_W, row_fire, 0)

    def row_drain(r, c):
        pltpu.make_async_copy(
            neg_hbm.at[pl.ds(0, EMB * NEG_W)],
            neg_hbm.at[pl.ds(0, EMB * NEG_W)],
            sem_n).wait()
        return c

    lax.fori_loop(0, B_PER_W, row_drain, 0)

    gcp.wait()
    pltpu.sync_copy(pos_v, pos_hbm.at[pl.ds(base, B_PER_W)])


_sc_call = functools.partial(
    pl.kernel,
    mesh=plsc.VectorSubcoreMesh(core_axis_name="c", subcore_axis_name="s"),
    out_type=(
        jax.ShapeDtypeStruct((BS, EMB * SIZE_PER_CLS), jnp.float32),
        jax.ShapeDtypeStruct((BS * EMB * NEG_W,), jnp.float32),
    ),
    scratch_types=[
        pltpu.VMEM((EMB, N_CLASSES * NEG_PER_CLS), jnp.float32),
        pltpu.VMEM((B_PER_W + 16,), jnp.int32),
        pltpu.VMEM((B_PER_W, EMB * SIZE_PER_CLS), jnp.float32),
        pltpu.SemaphoreType.DMA,
        pltpu.SemaphoreType.DMA,
    ],
    compiler_params=pltpu.CompilerParams(use_tc_tiling_on_sc=False),
)(_sc_body)


def kernel(labels, q):
    labels = labels.astype(jnp.int32)
    q_sel = jnp.stack([q[:, :, p] for p in PERM], axis=-1)
    full = jnp.transpose(q_sel, (1, 0, 2)).reshape(EMB, N_CLASSES * NEG_PER_CLS)
    q2 = q.reshape(N_CLASSES, EMB * SIZE_PER_CLS)
    pos, neg = _sc_call(labels, q2, full)
    return pos.reshape(BS, EMB, SIZE_PER_CLS), neg.reshape(BS, EMB, NEG_W)


# speedup vs baseline: 6.5041x; 1.3334x over previous
import functools
import jax
import jax.numpy as jnp
from jax import lax
from jax.experimental import pallas as pl
from jax.experimental.pallas import tpu as pltpu
from jax.experimental.pallas import tpu_sc as plsc

SIZE_PER_CLS = 32
N_CLASSES = 100
EMB = 64
NEG_PER_CLS = 8
BS = 512
PERM = (31, 7, 4, 29, 16, 19, 2, 5)

NC = 2
NS = 16
NW = NC * NS
B_PER_W = BS // NW
NEG_C = N_CLASSES - 1
NEG_W = NEG_C * NEG_PER_CLS
ROWS_PER_STEP = 4


def _pos_body(labels_hbm, q2_hbm, pos_hbm, lbl_v, pos_v, sem_g):
    wid = lax.axis_index("s") * NC + lax.axis_index("c")
    base = wid * B_PER_W
    pltpu.sync_copy(labels_hbm.at[pl.ds(base, B_PER_W)], lbl_v)
    gcp = pltpu.make_async_copy(q2_hbm.at[lbl_v], pos_v, sem_g)
    gcp.start()
    gcp.wait()
    pltpu.sync_copy(pos_v, pos_hbm.at[pl.ds(base, B_PER_W)])


_pos_call = functools.partial(
    pl.kernel,
    mesh=plsc.VectorSubcoreMesh(core_axis_name="c", subcore_axis_name="s"),
    out_type=jax.ShapeDtypeStruct((BS, EMB * SIZE_PER_CLS), jnp.float32),
    scratch_types=[
        pltpu.VMEM((B_PER_W,), jnp.int32),
        pltpu.VMEM((B_PER_W, EMB * SIZE_PER_CLS), jnp.float32),
        pltpu.SemaphoreType.DMA,
    ],
    compiler_params=pltpu.CompilerParams(use_tc_tiling_on_sc=False),
)(_pos_body)


def _neg_body(labels_smem, full_ref, out_ref):
    i = pl.program_id(0)
    lo = full_ref[:, : NEG_W]
    hi = full_ref[:, NEG_PER_CLS:]
    col = lax.broadcasted_iota(jnp.int32, (EMB, NEG_W), 1)
    for r in range(ROWS_PER_STEP):
        lb = labels_smem[i * ROWS_PER_STEP + r]
        out_ref[r] = jnp.where(col < NEG_PER_CLS * lb, lo, hi)


def _neg_call(labels, full):
    grid_spec = pltpu.PrefetchScalarGridSpec(
        num_scalar_prefetch=1,
        grid=(BS // ROWS_PER_STEP,),
        in_specs=[
            pl.BlockSpec((EMB, N_CLASSES * NEG_PER_CLS), lambda i, lbl: (0, 0)),
        ],
        out_specs=pl.BlockSpec(
            (ROWS_PER_STEP, EMB, NEG_W), lambda i, lbl: (i, 0, 0)),
    )
    return pl.pallas_call(
        _neg_body,
        grid_spec=grid_spec,
        out_shape=jax.ShapeDtypeStruct((BS, EMB, NEG_W), jnp.float32),
    )(labels, full)


def kernel(labels, q):
    labels = labels.astype(jnp.int32)
    q_sel = jnp.stack([q[:, :, p] for p in PERM], axis=-1)
    full = jnp.transpose(q_sel, (1, 0, 2)).reshape(EMB, N_CLASSES * NEG_PER_CLS)
    q2 = q.reshape(N_CLASSES, EMB * SIZE_PER_CLS)
    pos = _pos_call(labels, q2)
    neg = _neg_call(labels, full)
    return pos.reshape(BS, EMB, SIZE_PER_CLS), neg


# speedup vs baseline: 7.3343x; 1.1276x over previous
import functools
import jax
import jax.numpy as jnp
from jax import lax
from jax.experimental import pallas as pl
from jax.experimental.pallas import tpu as pltpu
from jax.experimental.pallas import tpu_sc as plsc

SIZE_PER_CLS = 32
N_CLASSES = 100
EMB = 64
NEG_PER_CLS = 8
BS = 512
PERM = (31, 7, 4, 29, 16, 19, 2, 5)

NC = 2
NS = 16
NW = NC * NS
B_PER_W = BS // NW
NEG_C = N_CLASSES - 1
NEG_W = NEG_C * NEG_PER_CLS
ROWS_PER_STEP = 8


def _pos_body(labels_hbm, q2_hbm, pos_hbm, lbl_v, pos_v, sem_g):
    wid = lax.axis_index("s") * NC + lax.axis_index("c")
    base = wid * B_PER_W
    pltpu.sync_copy(labels_hbm.at[pl.ds(base, B_PER_W)], lbl_v)
    gcp = pltpu.make_async_copy(q2_hbm.at[lbl_v], pos_v, sem_g)
    gcp.start()
    gcp.wait()
    pltpu.sync_copy(pos_v, pos_hbm.at[pl.ds(base, B_PER_W)])


_pos_call = functools.partial(
    pl.kernel,
    mesh=plsc.VectorSubcoreMesh(core_axis_name="c", subcore_axis_name="s"),
    out_type=jax.ShapeDtypeStruct((BS, EMB * SIZE_PER_CLS), jnp.float32),
    scratch_types=[
        pltpu.VMEM((B_PER_W,), jnp.int32),
        pltpu.VMEM((B_PER_W, EMB * SIZE_PER_CLS), jnp.float32),
        pltpu.SemaphoreType.DMA,
    ],
    compiler_params=pltpu.CompilerParams(use_tc_tiling_on_sc=False),
)(_pos_body)


def _neg_body(labels_smem, full_ref, out_ref):
    i = pl.program_id(0)
    lo = full_ref[:, : NEG_W]
    hi = full_ref[:, NEG_PER_CLS:]
    col = lax.broadcasted_iota(jnp.int32, (EMB, NEG_W), 1)
    for r in range(ROWS_PER_STEP):
        lb = labels_smem[i * ROWS_PER_STEP + r]
        out_ref[r] = jnp.where(col < NEG_PER_CLS * lb, lo, hi)


def _neg_call(labels, full):
    grid_spec = pltpu.PrefetchScalarGridSpec(
        num_scalar_prefetch=1,
        grid=(BS // ROWS_PER_STEP,),
        in_specs=[
            pl.BlockSpec((EMB, N_CLASSES * NEG_PER_CLS), lambda i, lbl: (0, 0)),
        ],
        out_specs=pl.BlockSpec(
            (ROWS_PER_STEP, EMB, NEG_W), lambda i, lbl: (i, 0, 0)),
    )
    return pl.pallas_call(
        _neg_body,
        grid_spec=grid_spec,
        out_shape=jax.ShapeDtypeStruct((BS, EMB, NEG_W), jnp.float32),
    )(labels, full)


def kernel(labels, q):
    labels = labels.astype(jnp.int32)
    q_sel = jnp.stack([q[:, :, p] for p in PERM], axis=-1)
    full = jnp.transpose(q_sel, (1, 0, 2)).reshape(EMB, N_CLASSES * NEG_PER_CLS)
    q2 = q.reshape(N_CLASSES, EMB * SIZE_PER_CLS)
    pos = _pos_call(labels, q2)
    neg = _neg_call(labels, full)
    return pos.reshape(BS, EMB, SIZE_PER_CLS), neg


# speedup vs baseline: 7.7643x; 1.0586x over previous
import functools
import jax
import jax.numpy as jnp
from jax import lax
from jax.experimental import pallas as pl
from jax.experimental.pallas import tpu as pltpu
from jax.experimental.pallas import tpu_sc as plsc

SIZE_PER_CLS = 32
N_CLASSES = 100
EMB = 64
NEG_PER_CLS = 8
BS = 512
PERM = (31, 7, 4, 29, 16, 19, 2, 5)

NC = 2
NS = 16
NW = NC * NS
B_PER_W = BS // NW
NEG_C = N_CLASSES - 1
NEG_W = NEG_C * NEG_PER_CLS
ROWS_PER_STEP = 16


def _pos_body(labels_hbm, q2_hbm, pos_hbm, lbl_v, pos_v, sem_g):
    wid = lax.axis_index("s") * NC + lax.axis_index("c")
    base = wid * B_PER_W
    pltpu.sync_copy(labels_hbm.at[pl.ds(base, B_PER_W)], lbl_v)
    gcp = pltpu.make_async_copy(q2_hbm.at[lbl_v], pos_v, sem_g)
    gcp.start()
    gcp.wait()
    pltpu.sync_copy(pos_v, pos_hbm.at[pl.ds(base, B_PER_W)])


_pos_call = functools.partial(
    pl.kernel,
    mesh=plsc.VectorSubcoreMesh(core_axis_name="c", subcore_axis_name="s"),
    out_type=jax.ShapeDtypeStruct((BS, EMB * SIZE_PER_CLS), jnp.float32),
    scratch_types=[
        pltpu.VMEM((B_PER_W,), jnp.int32),
        pltpu.VMEM((B_PER_W, EMB * SIZE_PER_CLS), jnp.float32),
        pltpu.SemaphoreType.DMA,
    ],
    compiler_params=pltpu.CompilerParams(use_tc_tiling_on_sc=False),
)(_pos_body)


def _neg_body(labels_smem, full_ref, out_ref):
    i = pl.program_id(0)
    lo = full_ref[:, : NEG_W]
    hi = full_ref[:, NEG_PER_CLS:]
    col = lax.broadcasted_iota(jnp.int32, (EMB, NEG_W), 1)
    for r in range(ROWS_PER_STEP):
        lb = labels_smem[i * ROWS_PER_STEP + r]
        out_ref[r] = jnp.where(col < NEG_PER_CLS * lb, lo, hi)


def _neg_call(labels, full):
    grid_spec = pltpu.PrefetchScalarGridSpec(
        num_scalar_prefetch=1,
        grid=(BS // ROWS_PER_STEP,),
        in_specs=[
            pl.BlockSpec((EMB, N_CLASSES * NEG_PER_CLS), lambda i, lbl: (0, 0)),
        ],
        out_specs=pl.BlockSpec(
            (ROWS_PER_STEP, EMB, NEG_W), lambda i, lbl: (i, 0, 0)),
    )
    return pl.pallas_call(
        _neg_body,
        grid_spec=grid_spec,
        out_shape=jax.ShapeDtypeStruct((BS, EMB, NEG_W), jnp.float32),
    )(labels, full)


def kernel(labels, q):
    labels = labels.astype(jnp.int32)
    q_sel = jnp.stack([q[:, :, p] for p in PERM], axis=-1)
    full = jnp.transpose(q_sel, (1, 0, 2)).reshape(EMB, N_CLASSES * NEG_PER_CLS)
    q2 = q.reshape(N_CLASSES, EMB * SIZE_PER_CLS)
    pos = _pos_call(labels, q2)
    neg = _neg_call(labels, full)
    return pos.reshape(BS, EMB, SIZE_PER_CLS), neg


# speedup vs baseline: 7.8036x; 1.0051x over previous
import functools
import jax
import jax.numpy as jnp
from jax import lax
from jax.experimental import pallas as pl
from jax.experimental.pallas import tpu as pltpu
from jax.experimental.pallas import tpu_sc as plsc

SIZE_PER_CLS = 32
N_CLASSES = 100
EMB = 64
NEG_PER_CLS = 8
BS = 512
PERM = (31, 7, 4, 29, 16, 19, 2, 5)

NC = 2
NS = 16
NW = NC * NS
B_PER_W = BS // NW
NEG_C = N_CLASSES - 1
NEG_W = NEG_C * NEG_PER_CLS
ROWS_PER_STEP = 32


def _pos_body(labels_hbm, q2_hbm, pos_hbm, lbl_v, pos_v, sem_g):
    wid = lax.axis_index("s") * NC + lax.axis_index("c")
    base = wid * B_PER_W
    pltpu.sync_copy(labels_hbm.at[pl.ds(base, B_PER_W)], lbl_v)
    gcp = pltpu.make_async_copy(q2_hbm.at[lbl_v], pos_v, sem_g)
    gcp.start()
    gcp.wait()
    pltpu.sync_copy(pos_v, pos_hbm.at[pl.ds(base, B_PER_W)])


_pos_call = functools.partial(
    pl.kernel,
    mesh=plsc.VectorSubcoreMesh(core_axis_name="c", subcore_axis_name="s"),
    out_type=jax.ShapeDtypeStruct((BS, EMB * SIZE_PER_CLS), jnp.float32),
    scratch_types=[
        pltpu.VMEM((B_PER_W,), jnp.int32),
        pltpu.VMEM((B_PER_W, EMB * SIZE_PER_CLS), jnp.float32),
        pltpu.SemaphoreType.DMA,
    ],
    compiler_params=pltpu.CompilerParams(use_tc_tiling_on_sc=False),
)(_pos_body)


def _neg_body(labels_smem, full_ref, out_ref):
    i = pl.program_id(0)
    lo = full_ref[:, : NEG_W]
    hi = full_ref[:, NEG_PER_CLS:]
    col = lax.broadcasted_iota(jnp.int32, (EMB, NEG_W), 1)
    for r in range(ROWS_PER_STEP):
        lb = labels_smem[i * ROWS_PER_STEP + r]
        out_ref[r] = jnp.where(col < NEG_PER_CLS * lb, lo, hi)


def _neg_call(labels, full):
    grid_spec = pltpu.PrefetchScalarGridSpec(
        num_scalar_prefetch=1,
        grid=(BS // ROWS_PER_STEP,),
        in_specs=[
            pl.BlockSpec((EMB, N_CLASSES * NEG_PER_CLS), lambda i, lbl: (0, 0)),
        ],
        out_specs=pl.BlockSpec(
            (ROWS_PER_STEP, EMB, NEG_W), lambda i, lbl: (i, 0, 0)),
    )
    return pl.pallas_call(
        _neg_body,
        grid_spec=grid_spec,
        out_shape=jax.ShapeDtypeStruct((BS, EMB, NEG_W), jnp.float32),
    )(labels, full)


def kernel(labels, q):
    labels = labels.astype(jnp.int32)
    q_sel = jnp.stack([q[:, :, p] for p in PERM], axis=-1)
    full = jnp.transpose(q_sel, (1, 0, 2)).reshape(EMB, N_CLASSES * NEG_PER_CLS)
    q2 = q.reshape(N_CLASSES, EMB * SIZE_PER_CLS)
    pos = _pos_call(labels, q2)
    neg = _neg_call(labels, full)
    return pos.reshape(BS, EMB, SIZE_PER_CLS), neg
